# Initial kernel scaffold; baseline (speedup 1.0000x reference)
import jax
import jax.numpy as jnp
from jax.experimental import pallas as pl


def kernel(img, subsample_idx, height, width):
    raise NotImplementedError("write your pallas kernel here")


# speedup vs baseline: 4.0626x; 4.0626x over previous
import functools

import jax
import jax.numpy as jnp
from jax import lax
from jax.experimental import pallas as pl
from jax.experimental.pallas import tpu as pltpu
from jax.experimental.pallas import tpu_sc as plsc

_N_PIX = 512 * 512
_N_SUB = _N_PIX // 2
_N_WORKERS = 32
_PIX_PER = _N_PIX // _N_WORKERS
_N_PAD = 8192
_ICHUNK = 4096
_N_ICHUNKS = _N_SUB // _ICHUNK
_GCHUNK = 1024
_N_GCHUNKS = _PIX_PER // _GCHUNK
_GSUB = 128


def _sc_scatter(idx, imgT):
    mesh = plsc.VectorSubcoreMesh(core_axis_name="c", subcore_axis_name="s")

    @functools.partial(
        pl.kernel,
        mesh=mesh,
        out_type=jax.ShapeDtypeStruct((16, _N_PIX), jnp.float32),
        scratch_types=[
            pltpu.VMEM((_PIX_PER,), jnp.int32),
            pltpu.VMEM((_ICHUNK,), jnp.int32),
            pltpu.VMEM((_ICHUNK,), jnp.int32),
            pltpu.VMEM((_GCHUNK, 16), jnp.float32),
            pltpu.VMEM((_GCHUNK * 16,), jnp.float32),
            pltpu.SemaphoreType.DMA,
            pltpu.SemaphoreType.DMA,
            pltpu.SemaphoreType.DMA,
            pltpu.SemaphoreType.DMA,
        ],
        compiler_params=pltpu.CompilerParams(
            needs_layout_passes=False, use_tc_tiling_on_sc=False),
    )
    def body(idx_hbm, imgT_hbm, out_hbm, claim_v, ibuf0, ibuf1, gbuf, tbuf,
             isem0, isem1, gsem, osem):
        wid = lax.axis_index("s") * 2 + lax.axis_index("c")
        base = wid * _PIX_PER
        lanes = lax.iota(jnp.int32, 16)

        def init_body(v, _):
            claim_v[pl.ds(v * 16, 16)] = lanes + (_N_SUB + v * 16)
            return _
        lax.fori_loop(0, _PIX_PER // 16, init_body, 0)

        h0 = pltpu.async_copy(idx_hbm.at[pl.ds(0, _ICHUNK)], ibuf0, isem0)
        handles = [h0, None]
        for c in range(_N_ICHUNKS):
            par = c % 2
            if c + 1 < _N_ICHUNKS:
                nxt = (c + 1) % 2
                handles[nxt] = pltpu.async_copy(
                    idx_hbm.at[pl.ds((c + 1) * _ICHUNK, _ICHUNK)],
                    ibuf1 if nxt else ibuf0, isem1 if nxt else isem0)
            handles[par].wait()
            ibuf = ibuf1 if par else ibuf0

            def scan_body(v, _, c=c, ibuf=ibuf):
                iv = ibuf[pl.ds(v * 16, 16)]
                m = (iv >= base) & (iv < base + _PIX_PER)
                local = (iv - base) & (_PIX_PER - 1)
                jv = lanes + (c * _ICHUNK + v * 16)
                plsc.store_scatter(claim_v, [local], jv, mask=m)
                return _
            lax.fori_loop(0, _ICHUNK // 16, scan_body, 0)

        lanes_sc = lanes * _GCHUNK
        for c2 in range(_N_GCHUNKS):
            for g in range(_GCHUNK // _GSUB):
                pltpu.async_copy(
                    imgT_hbm.at[claim_v.at[pl.ds(c2 * _GCHUNK + g * _GSUB,
                                                 _GSUB)]],
                    gbuf.at[pl.ds(g * _GSUB, _GSUB)], gsem)
            for g in range(_GCHUNK // _GSUB):
                pltpu.make_async_copy(
                    imgT_hbm.at[claim_v.at[pl.ds(c2 * _GCHUNK + g * _GSUB,
                                                 _GSUB)]],
                    gbuf.at[pl.ds(g * _GSUB, _GSUB)], gsem).wait()

            def tr_body(p, _):
                row = gbuf[p]
                plsc.store_scatter(tbuf, [lanes_sc + p], row)
                return _
            lax.fori_loop(0, _GCHUNK, tr_body, 0)

            for b in range(16):
                pltpu.async_copy(
                    tbuf.at[pl.ds(b * _GCHUNK, _GCHUNK)],
                    out_hbm.at[b, pl.ds(base + c2 * _GCHUNK, _GCHUNK)], osem)
            for b in range(16):
                pltpu.make_async_copy(
                    tbuf.at[pl.ds(b * _GCHUNK, _GCHUNK)],
                    out_hbm.at[b, pl.ds(base + c2 * _GCHUNK, _GCHUNK)],
                    osem).wait()

    return body(idx, imgT)


def kernel(img, subsample_idx, height, width):
    idx = (subsample_idx + (height - 512) + (width - 512)).astype(jnp.int32)
    imgT = jnp.concatenate(
        [img.T, jnp.zeros((_N_PAD, img.shape[0]), img.dtype)], axis=0)
    out = _sc_scatter(idx, imgT)
    return out.reshape(img.shape[0], 1, 512, 512)


# speedup vs baseline: 4.3125x; 1.0615x over previous
import functools

import jax
import jax.numpy as jnp
from jax import lax
from jax.experimental import pallas as pl
from jax.experimental.pallas import tpu as pltpu
from jax.experimental.pallas import tpu_sc as plsc

_N_PIX = 512 * 512
_N_SUB = _N_PIX // 2
_N_WORKERS = 32
_PIX_PER = _N_PIX // _N_WORKERS
_N_PAD = 8192
_ICHUNK = 4096
_N_ICHUNKS = _N_SUB // _ICHUNK
_GCHUNK = 1024
_N_GCHUNKS = _PIX_PER // _GCHUNK
_GSUB = 128


def _sc_scatter(idx, imgT):
    mesh = plsc.VectorSubcoreMesh(core_axis_name="c", subcore_axis_name="s")

    @functools.partial(
        pl.kernel,
        mesh=mesh,
        out_type=jax.ShapeDtypeStruct((16, _N_PIX), jnp.float32),
        scratch_types=[
            pltpu.VMEM((_PIX_PER,), jnp.int32),
            pltpu.VMEM((_ICHUNK,), jnp.int32),
            pltpu.VMEM((_ICHUNK,), jnp.int32),
            pltpu.VMEM((_GCHUNK, 16), jnp.float32),
            pltpu.VMEM((_GCHUNK, 16), jnp.float32),
            pltpu.VMEM((_GCHUNK * 16,), jnp.float32),
            pltpu.VMEM((_GCHUNK * 16,), jnp.float32),
            pltpu.SemaphoreType.DMA,
            pltpu.SemaphoreType.DMA,
            pltpu.SemaphoreType.DMA,
            pltpu.SemaphoreType.DMA,
            pltpu.SemaphoreType.DMA,
            pltpu.SemaphoreType.DMA,
        ],
        compiler_params=pltpu.CompilerParams(
            needs_layout_passes=False, use_tc_tiling_on_sc=False),
    )
    def body(idx_hbm, imgT_hbm, out_hbm, claim_v, ibuf0, ibuf1,
             gbuf0, gbuf1, tbuf0, tbuf1,
             isem0, isem1, gsem0, gsem1, osem0, osem1):
        wid = lax.axis_index("s") * 2 + lax.axis_index("c")
        base = wid * _PIX_PER
        lanes = lax.iota(jnp.int32, 16)

        def init_body(v, _):
            claim_v[pl.ds(v * 16, 16)] = lanes + (_N_SUB + v * 16)
            return _
        lax.fori_loop(0, _PIX_PER // 16, init_body, 0)

        ibufs = [ibuf0, ibuf1]
        isems = [isem0, isem1]
        pltpu.async_copy(idx_hbm.at[pl.ds(0, _ICHUNK)], ibuf0, isem0)
        for c in range(_N_ICHUNKS):
            par = c % 2
            if c + 1 < _N_ICHUNKS:
                nxt = (c + 1) % 2
                pltpu.async_copy(
                    idx_hbm.at[pl.ds((c + 1) * _ICHUNK, _ICHUNK)],
                    ibufs[nxt], isems[nxt])
            pltpu.make_async_copy(
                idx_hbm.at[pl.ds(c * _ICHUNK, _ICHUNK)],
                ibufs[par], isems[par]).wait()
            ibuf = ibufs[par]

            def scan_body(u, _, c=c, ibuf=ibuf):
                for k in range(4):
                    v = u * 4 + k
                    iv = ibuf[pl.ds(v * 16, 16)]
                    m = (iv >= base) & (iv < base + _PIX_PER)
                    local = (iv - base) & (_PIX_PER - 1)
                    jv = lanes + (c * _ICHUNK + v * 16)
                    plsc.store_scatter(claim_v, [local], jv, mask=m)
                return _
            lax.fori_loop(0, _ICHUNK // 64, scan_body, 0)

        lanes_sc = lanes * _GCHUNK
        gbufs = [gbuf0, gbuf1]
        gsems = [gsem0, gsem1]
        tbufs = [tbuf0, tbuf1]
        osems = [osem0, osem1]

        def fire_gathers(c2):
            par = c2 % 2
            for g in range(_GCHUNK // _GSUB):
                pltpu.async_copy(
                    imgT_hbm.at[claim_v.at[pl.ds(c2 * _GCHUNK + g * _GSUB,
                                                 _GSUB)]],
                    gbufs[par].at[pl.ds(g * _GSUB, _GSUB)], gsems[par])

        def drain_gathers(c2):
            par = c2 % 2
            for g in range(_GCHUNK // _GSUB):
                pltpu.make_async_copy(
                    imgT_hbm.at[claim_v.at[pl.ds(c2 * _GCHUNK + g * _GSUB,
                                                 _GSUB)]],
                    gbufs[par].at[pl.ds(g * _GSUB, _GSUB)], gsems[par]).wait()

        def fire_out(c2):
            par = c2 % 2
            for b in range(16):
                pltpu.async_copy(
                    tbufs[par].at[pl.ds(b * _GCHUNK, _GCHUNK)],
                    out_hbm.at[b, pl.ds(base + c2 * _GCHUNK, _GCHUNK)],
                    osems[par])

        def drain_out(c2):
            par = c2 % 2
            for b in range(16):
                pltpu.make_async_copy(
                    tbufs[par].at[pl.ds(b * _GCHUNK, _GCHUNK)],
                    out_hbm.at[b, pl.ds(base + c2 * _GCHUNK, _GCHUNK)],
                    osems[par]).wait()

        fire_gathers(0)
        for c2 in range(_N_GCHUNKS):
            if c2 + 1 < _N_GCHUNKS:
                fire_gathers(c2 + 1)
            drain_gathers(c2)
            if c2 >= 2:
                drain_out(c2 - 2)

            gbuf = gbufs[c2 % 2]
            tbuf = tbufs[c2 % 2]

            def tr_body(u, _, gbuf=gbuf, tbuf=tbuf):
                for k in range(4):
                    p = u * 4 + k
                    row = gbuf[p]
                    plsc.store_scatter(tbuf, [lanes_sc + p], row)
                return _
            lax.fori_loop(0, _GCHUNK // 4, tr_body, 0)
            fire_out(c2)
        drain_out(_N_GCHUNKS - 2)
        drain_out(_N_GCHUNKS - 1)

    return body(idx, imgT)


def kernel(img, subsample_idx, height, width):
    idx = (subsample_idx + (height - 512) + (width - 512)).astype(jnp.int32)
    imgT = jnp.concatenate(
        [img.T, jnp.zeros((_N_PAD, img.shape[0]), img.dtype)], axis=0)
    out = _sc_scatter(idx, imgT)
    return out.reshape(img.shape[0], 1, 512, 512)
